# baseline (speedup 1.0000x reference)
import functools

import jax
import jax.numpy as jnp
from jax import lax
from jax.experimental import pallas as pl
from jax.experimental.pallas import tpu as pltpu
from jax.experimental.pallas import tpu_sc as plsc

NUM_EXPERTS = 3
N_LAYER = 48
N_EMBD = 4096
BATCH = 16384
N_ROWS = NUM_EXPERTS * N_LAYER

TC_N = 8192
SC_N = BATCH - TC_N

NUM_CORES = 2
NUM_SUBCORES = 16
LANES = 16
NUM_WORKERS = NUM_CORES * NUM_SUBCORES
B_PER_W = SC_N // NUM_WORKERS
F_DIRECT = 96
K = 8
CHUNKS = (B_PER_W - F_DIRECT) // K
NBUF = 2
IDX_PAD = B_PER_W + LANES

_mesh = plsc.VectorSubcoreMesh(core_axis_name="c", subcore_axis_name="s")


@functools.partial(
    pl.kernel,
    mesh=_mesh,
    out_type=jax.ShapeDtypeStruct((SC_N, N_EMBD), jnp.float32),
    scratch_types=[
        pltpu.VMEM((B_PER_W,), jnp.int32),
        pltpu.VMEM((B_PER_W,), jnp.int32),
        pltpu.VMEM((IDX_PAD,), jnp.int32),
        pltpu.VMEM((NBUF, K, N_EMBD), jnp.float32),
        pltpu.VMEM_SHARED((N_ROWS * N_EMBD,), jnp.float32),
        pltpu.SemaphoreType.DMA((NBUF,)),
        pltpu.SemaphoreType.DMA((NBUF,)),
        pltpu.SemaphoreType.DMA,
    ],
)
def _sc_lookup(table_hbm, task_hbm, layer_hbm, out_hbm,
               task_v, layer_v, idx_v, rows_v, table_sp, gsem, wsem, dsem):
    sid = lax.axis_index("s")
    wid = sid * NUM_CORES + lax.axis_index("c")
    base = wid * B_PER_W

    n_chunks = N_ROWS // 8
    csz = 8 * N_EMBD

    def stage(j):
        off = pl.multiple_of(j * csz, 8)
        pltpu.sync_copy(table_hbm.at[pl.ds(off, csz)],
                        table_sp.at[pl.ds(off, csz)])

    stage(sid)

    @pl.when(sid < n_chunks - NUM_SUBCORES)
    def _():
        stage(NUM_SUBCORES + sid)

    pltpu.sync_copy(task_hbm.at[pl.ds(base, B_PER_W)], task_v)
    pltpu.sync_copy(layer_hbm.at[pl.ds(base, B_PER_W)], layer_v)

    def fuse(i, carry):
        sl = pl.ds(i * LANES, LANES)
        idx_v[sl] = task_v[sl] * N_LAYER + layer_v[sl]
        return carry

    lax.fori_loop(0, B_PER_W // LANES, fuse, 0)
    plsc.subcore_barrier()

    for g in range(F_DIRECT // LANES):
        dvec = idx_v[pl.ds(g * LANES, LANES)]
        for k in range(LANES):
            p = g * LANES + k
            src = table_sp.at[
                pl.ds(pl.multiple_of(dvec[k] * N_EMBD, 8), N_EMBD)]
            pltpu.make_async_copy(src, out_hbm.at[base + p], dsem).start()

    def fill(c, b):
        vec = idx_v[pl.ds(pl.multiple_of(F_DIRECT + c * K, 8), LANES)]
        for k in range(K):
            src = table_sp.at[
                pl.ds(pl.multiple_of(vec[k] * N_EMBD, 8), N_EMBD)]
            pltpu.make_async_copy(src, rows_v.at[b, k], gsem.at[b]).start()

    def fill_wait(b):
        for k in range(K):
            pltpu.make_async_copy(table_sp.at[pl.ds(0, N_EMBD)],
                                  rows_v.at[b, k], gsem.at[b]).wait()

    def write_desc(c, b):
        off = pl.multiple_of(F_DIRECT + c * K, 8)
        return pltpu.make_async_copy(rows_v.at[b],
                                     out_hbm.at[pl.ds(base + off, K)],
                                     wsem.at[b])

    for b in range(NBUF):
        fill(b, b)

    def step(c, b):
        fill_wait(b)
        wr = write_desc(c, b)
        wr.start()
        wr.wait()

        @pl.when(c + NBUF < CHUNKS)
        def _():
            fill(c + NBUF, b)

    def outer(i, carry):
        for b in range(NBUF):
            step(i * NBUF + b, b)
        return carry

    full = CHUNKS // NBUF
    lax.fori_loop(0, full, outer, 0)
    for c in range(full * NBUF, CHUNKS):
        step(c, c % NBUF)

    for g in range(F_DIRECT // LANES):
        sl = pl.ds(0, LANES * N_EMBD)
        pltpu.make_async_copy(table_hbm.at[sl], table_sp.at[sl], dsem).wait()


TC_BLK = 256
TC_NBLK = TC_N // TC_BLK


def _tc_body(idx_ref, table_ref, out_ref):
    idx = idx_ref[0]
    rows = lax.broadcasted_iota(jnp.int32, (N_ROWS, TC_BLK), 0)
    one_hot = (rows == idx).astype(jnp.float32)
    out_ref[...] = lax.dot_general(
        one_hot, table_ref[...],
        dimension_numbers=(((0,), (0,)), ((), ())),
        preferred_element_type=jnp.float32)


_tc_lookup = pl.pallas_call(
    _tc_body,
    grid=(TC_NBLK,),
    in_specs=[
        pl.BlockSpec((1, 1, TC_BLK), lambda i: (i, 0, 0)),
        pl.BlockSpec((N_ROWS, N_EMBD), lambda i: (0, 0)),
    ],
    out_specs=pl.BlockSpec((TC_BLK, N_EMBD), lambda i: (i, 0)),
    out_shape=jax.ShapeDtypeStruct((TC_N, N_EMBD), jnp.float32),
)


def kernel(experts, task_idx, layer_idx):
    table2d = experts.reshape(N_ROWS, N_EMBD)
    flat_idx = (task_idx.astype(jnp.int32) * N_LAYER
                + layer_idx.astype(jnp.int32))
    tc_idx = flat_idx[:TC_N].reshape(TC_NBLK, 1, TC_BLK)
    tc_out = _tc_lookup(tc_idx, table2d)
    sc_out = _sc_lookup(table2d.reshape(N_ROWS * N_EMBD),
                        task_idx[TC_N:].astype(jnp.int32),
                        layer_idx[TC_N:].astype(jnp.int32))
    return jnp.concatenate([tc_out, sc_out], axis=0)


# speedup vs baseline: 1.6369x; 1.6369x over previous
import functools

import jax
import jax.numpy as jnp
from jax import lax
from jax.experimental import pallas as pl
from jax.experimental.pallas import tpu as pltpu
from jax.experimental.pallas import tpu_sc as plsc

NUM_EXPERTS = 3
N_LAYER = 48
N_EMBD = 4096
BATCH = 16384
N_ROWS = NUM_EXPERTS * N_LAYER

NUM_CORES = 2
NUM_SUBCORES = 16
LANES = 16
NUM_WORKERS = NUM_CORES * NUM_SUBCORES
B_PER_W = BATCH // NUM_WORKERS
F_DIRECT = 256
K = 8
CHUNKS = (B_PER_W - F_DIRECT) // K
NBUF = 2
IDX_PAD = B_PER_W + LANES

_mesh = plsc.VectorSubcoreMesh(core_axis_name="c", subcore_axis_name="s")


@functools.partial(
    pl.kernel,
    mesh=_mesh,
    out_type=jax.ShapeDtypeStruct((BATCH, N_EMBD), jnp.float32),
    scratch_types=[
        pltpu.VMEM((B_PER_W,), jnp.int32),
        pltpu.VMEM((B_PER_W,), jnp.int32),
        pltpu.VMEM((IDX_PAD,), jnp.int32),
        pltpu.VMEM((NBUF, K, N_EMBD), jnp.float32),
        pltpu.VMEM_SHARED((N_ROWS * N_EMBD,), jnp.float32),
        pltpu.SemaphoreType.DMA((NBUF,)),
        pltpu.SemaphoreType.DMA((NBUF,)),
        pltpu.SemaphoreType.DMA,
    ],
)
def _lookup_kernel(table_hbm, task_hbm, layer_hbm, out_hbm,
                   task_v, layer_v, idx_v, rows_v, table_sp, gsem, wsem, dsem):
    sid = lax.axis_index("s")
    wid = sid * NUM_CORES + lax.axis_index("c")
    base = wid * B_PER_W

    n_chunks = N_ROWS // 8
    csz = 8 * N_EMBD

    def stage(j):
        off = pl.multiple_of(j * csz, 8)
        pltpu.sync_copy(table_hbm.at[pl.ds(off, csz)],
                        table_sp.at[pl.ds(off, csz)])

    stage(sid)

    @pl.when(sid < n_chunks - NUM_SUBCORES)
    def _():
        stage(NUM_SUBCORES + sid)

    pltpu.sync_copy(task_hbm.at[pl.ds(base, B_PER_W)], task_v)
    pltpu.sync_copy(layer_hbm.at[pl.ds(base, B_PER_W)], layer_v)

    def fuse(i, carry):
        sl = pl.ds(i * LANES, LANES)
        idx_v[sl] = task_v[sl] * N_LAYER + layer_v[sl]
        return carry

    lax.fori_loop(0, B_PER_W // LANES, fuse, 0)
    plsc.subcore_barrier()

    for g in range(F_DIRECT // LANES):
        dvec = idx_v[pl.ds(g * LANES, LANES)]
        for k in range(LANES):
            p = g * LANES + k
            src = table_sp.at[
                pl.ds(pl.multiple_of(dvec[k] * N_EMBD, 8), N_EMBD)]
            pltpu.make_async_copy(src, out_hbm.at[base + p], dsem).start()

    def fill(c, b):
        vec = idx_v[pl.ds(pl.multiple_of(F_DIRECT + c * K, 8), LANES)]
        for k in range(K):
            src = table_sp.at[
                pl.ds(pl.multiple_of(vec[k] * N_EMBD, 8), N_EMBD)]
            pltpu.make_async_copy(src, rows_v.at[b, k], gsem.at[b]).start()

    def fill_wait(b):
        for k in range(K):
            pltpu.make_async_copy(table_sp.at[pl.ds(0, N_EMBD)],
                                  rows_v.at[b, k], gsem.at[b]).wait()

    def write_desc(c, b):
        off = pl.multiple_of(F_DIRECT + c * K, 8)
        return pltpu.make_async_copy(rows_v.at[b],
                                     out_hbm.at[pl.ds(base + off, K)],
                                     wsem.at[b])

    for b in range(NBUF):
        fill(b, b)

    def step(c, b):
        fill_wait(b)
        wr = write_desc(c, b)
        wr.start()
        wr.wait()

        @pl.when(c + NBUF < CHUNKS)
        def _():
            fill(c + NBUF, b)

    def outer(i, carry):
        for b in range(NBUF):
            step(i * NBUF + b, b)
        return carry

    full = CHUNKS // NBUF
    lax.fori_loop(0, full, outer, 0)
    for c in range(full * NBUF, CHUNKS):
        step(c, c % NBUF)

    for g in range(F_DIRECT // LANES):
        sl = pl.ds(0, LANES * N_EMBD)
        pltpu.make_async_copy(table_hbm.at[sl], table_sp.at[sl], dsem).wait()


def kernel(experts, task_idx, layer_idx):
    table = experts.reshape(N_ROWS * N_EMBD)
    return _lookup_kernel(table,
                          task_idx.astype(jnp.int32),
                          layer_idx.astype(jnp.int32))


# speedup vs baseline: 1.6540x; 1.0104x over previous
import functools

import jax
import jax.numpy as jnp
from jax import lax
from jax.experimental import pallas as pl
from jax.experimental.pallas import tpu as pltpu
from jax.experimental.pallas import tpu_sc as plsc

NUM_EXPERTS = 3
N_LAYER = 48
N_EMBD = 4096
BATCH = 16384
N_ROWS = NUM_EXPERTS * N_LAYER

NUM_CORES = 2
NUM_SUBCORES = 16
LANES = 16
NUM_WORKERS = NUM_CORES * NUM_SUBCORES
B_PER_W = BATCH // NUM_WORKERS
F_DIRECT = 320
K = 8
CHUNKS = (B_PER_W - F_DIRECT) // K
NBUF = 2
IDX_PAD = B_PER_W + LANES

_mesh = plsc.VectorSubcoreMesh(core_axis_name="c", subcore_axis_name="s")


@functools.partial(
    pl.kernel,
    mesh=_mesh,
    out_type=jax.ShapeDtypeStruct((BATCH, N_EMBD), jnp.float32),
    scratch_types=[
        pltpu.VMEM((B_PER_W,), jnp.int32),
        pltpu.VMEM((B_PER_W,), jnp.int32),
        pltpu.VMEM((IDX_PAD,), jnp.int32),
        pltpu.VMEM((NBUF, K, N_EMBD), jnp.float32),
        pltpu.VMEM_SHARED((N_ROWS * N_EMBD,), jnp.float32),
        pltpu.SemaphoreType.DMA((NBUF,)),
        pltpu.SemaphoreType.DMA((NBUF,)),
        pltpu.SemaphoreType.DMA,
    ],
)
def _lookup_kernel(table_hbm, task_hbm, layer_hbm, out_hbm,
                   task_v, layer_v, idx_v, rows_v, table_sp, gsem, wsem, dsem):
    sid = lax.axis_index("s")
    wid = sid * NUM_CORES + lax.axis_index("c")
    base = wid * B_PER_W

    n_chunks = N_ROWS // 8
    csz = 8 * N_EMBD

    def stage(j):
        off = pl.multiple_of(j * csz, 8)
        pltpu.sync_copy(table_hbm.at[pl.ds(off, csz)],
                        table_sp.at[pl.ds(off, csz)])

    stage(sid)

    @pl.when(sid < n_chunks - NUM_SUBCORES)
    def _():
        stage(NUM_SUBCORES + sid)

    pltpu.sync_copy(task_hbm.at[pl.ds(base, B_PER_W)], task_v)
    pltpu.sync_copy(layer_hbm.at[pl.ds(base, B_PER_W)], layer_v)

    def fuse(i, carry):
        sl = pl.ds(i * LANES, LANES)
        idx_v[sl] = task_v[sl] * N_LAYER + layer_v[sl]
        return carry

    lax.fori_loop(0, B_PER_W // LANES, fuse, 0)
    plsc.subcore_barrier()

    for g in range(F_DIRECT // LANES):
        dvec = idx_v[pl.ds(g * LANES, LANES)]
        for k in range(LANES):
            p = g * LANES + k
            src = table_sp.at[
                pl.ds(pl.multiple_of(dvec[k] * N_EMBD, 8), N_EMBD)]
            pltpu.make_async_copy(src, out_hbm.at[base + p], dsem).start()

    def fill(c, b):
        vec = idx_v[pl.ds(pl.multiple_of(F_DIRECT + c * K, 8), LANES)]
        for k in range(K):
            src = table_sp.at[
                pl.ds(pl.multiple_of(vec[k] * N_EMBD, 8), N_EMBD)]
            pltpu.make_async_copy(src, rows_v.at[b, k], gsem.at[b]).start()

    def fill_wait(b):
        for k in range(K):
            pltpu.make_async_copy(table_sp.at[pl.ds(0, N_EMBD)],
                                  rows_v.at[b, k], gsem.at[b]).wait()

    def write_desc(c, b):
        off = pl.multiple_of(F_DIRECT + c * K, 8)
        return pltpu.make_async_copy(rows_v.at[b],
                                     out_hbm.at[pl.ds(base + off, K)],
                                     wsem.at[b])

    for b in range(NBUF):
        fill(b, b)

    def step(c, b):
        fill_wait(b)
        wr = write_desc(c, b)
        wr.start()
        wr.wait()

        @pl.when(c + NBUF < CHUNKS)
        def _():
            fill(c + NBUF, b)

    def outer(i, carry):
        for b in range(NBUF):
            step(i * NBUF + b, b)
        return carry

    full = CHUNKS // NBUF
    lax.fori_loop(0, full, outer, 0)
    for c in range(full * NBUF, CHUNKS):
        step(c, c % NBUF)

    for g in range(F_DIRECT // LANES):
        sl = pl.ds(0, LANES * N_EMBD)
        pltpu.make_async_copy(table_hbm.at[sl], table_sp.at[sl], dsem).wait()


def kernel(experts, task_idx, layer_idx):
    table = experts.reshape(N_ROWS * N_EMBD)
    return _lookup_kernel(table,
                          task_idx.astype(jnp.int32),
                          layer_idx.astype(jnp.int32))


# speedup vs baseline: 1.6695x; 1.0094x over previous
import functools

import jax
import jax.numpy as jnp
from jax import lax
from jax.experimental import pallas as pl
from jax.experimental.pallas import tpu as pltpu
from jax.experimental.pallas import tpu_sc as plsc

NUM_EXPERTS = 3
N_LAYER = 48
N_EMBD = 4096
BATCH = 16384
N_ROWS = NUM_EXPERTS * N_LAYER

NUM_CORES = 2
NUM_SUBCORES = 16
LANES = 16
NUM_WORKERS = NUM_CORES * NUM_SUBCORES
B_PER_W = BATCH // NUM_WORKERS
F_DIRECT = 384
K = 8
CHUNKS = (B_PER_W - F_DIRECT) // K
NBUF = 2
IDX_PAD = B_PER_W + LANES

_mesh = plsc.VectorSubcoreMesh(core_axis_name="c", subcore_axis_name="s")


@functools.partial(
    pl.kernel,
    mesh=_mesh,
    out_type=jax.ShapeDtypeStruct((BATCH, N_EMBD), jnp.float32),
    scratch_types=[
        pltpu.VMEM((B_PER_W,), jnp.int32),
        pltpu.VMEM((B_PER_W,), jnp.int32),
        pltpu.VMEM((IDX_PAD,), jnp.int32),
        pltpu.VMEM((NBUF, K, N_EMBD), jnp.float32),
        pltpu.VMEM_SHARED((N_ROWS * N_EMBD,), jnp.float32),
        pltpu.SemaphoreType.DMA((NBUF,)),
        pltpu.SemaphoreType.DMA((NBUF,)),
        pltpu.SemaphoreType.DMA,
    ],
)
def _lookup_kernel(table_hbm, task_hbm, layer_hbm, out_hbm,
                   task_v, layer_v, idx_v, rows_v, table_sp, gsem, wsem, dsem):
    sid = lax.axis_index("s")
    wid = sid * NUM_CORES + lax.axis_index("c")
    base = wid * B_PER_W

    n_chunks = N_ROWS // 8
    csz = 8 * N_EMBD

    def stage(j):
        off = pl.multiple_of(j * csz, 8)
        pltpu.sync_copy(table_hbm.at[pl.ds(off, csz)],
                        table_sp.at[pl.ds(off, csz)])

    stage(sid)

    @pl.when(sid < n_chunks - NUM_SUBCORES)
    def _():
        stage(NUM_SUBCORES + sid)

    pltpu.sync_copy(task_hbm.at[pl.ds(base, B_PER_W)], task_v)
    pltpu.sync_copy(layer_hbm.at[pl.ds(base, B_PER_W)], layer_v)

    def fuse(i, carry):
        sl = pl.ds(i * LANES, LANES)
        idx_v[sl] = task_v[sl] * N_LAYER + layer_v[sl]
        return carry

    lax.fori_loop(0, B_PER_W // LANES, fuse, 0)
    plsc.subcore_barrier()

    for g in range(F_DIRECT // LANES):
        dvec = idx_v[pl.ds(g * LANES, LANES)]
        for k in range(LANES):
            p = g * LANES + k
            src = table_sp.at[
                pl.ds(pl.multiple_of(dvec[k] * N_EMBD, 8), N_EMBD)]
            pltpu.make_async_copy(src, out_hbm.at[base + p], dsem).start()

    def fill(c, b):
        vec = idx_v[pl.ds(pl.multiple_of(F_DIRECT + c * K, 8), LANES)]
        for k in range(K):
            src = table_sp.at[
                pl.ds(pl.multiple_of(vec[k] * N_EMBD, 8), N_EMBD)]
            pltpu.make_async_copy(src, rows_v.at[b, k], gsem.at[b]).start()

    def fill_wait(b):
        for k in range(K):
            pltpu.make_async_copy(table_sp.at[pl.ds(0, N_EMBD)],
                                  rows_v.at[b, k], gsem.at[b]).wait()

    def write_desc(c, b):
        off = pl.multiple_of(F_DIRECT + c * K, 8)
        return pltpu.make_async_copy(rows_v.at[b],
                                     out_hbm.at[pl.ds(base + off, K)],
                                     wsem.at[b])

    for b in range(NBUF):
        fill(b, b)

    def step(c, b):
        fill_wait(b)
        wr = write_desc(c, b)
        wr.start()
        wr.wait()

        @pl.when(c + NBUF < CHUNKS)
        def _():
            fill(c + NBUF, b)

    def outer(i, carry):
        for b in range(NBUF):
            step(i * NBUF + b, b)
        return carry

    full = CHUNKS // NBUF
    lax.fori_loop(0, full, outer, 0)
    for c in range(full * NBUF, CHUNKS):
        step(c, c % NBUF)

    for g in range(F_DIRECT // LANES):
        sl = pl.ds(0, LANES * N_EMBD)
        pltpu.make_async_copy(table_hbm.at[sl], table_sp.at[sl], dsem).wait()


def kernel(experts, task_idx, layer_idx):
    table = experts.reshape(N_ROWS * N_EMBD)
    return _lookup_kernel(table,
                          task_idx.astype(jnp.int32),
                          layer_idx.astype(jnp.int32))


# speedup vs baseline: 1.6883x; 1.0112x over previous
import functools

import jax
import jax.numpy as jnp
from jax import lax
from jax.experimental import pallas as pl
from jax.experimental.pallas import tpu as pltpu
from jax.experimental.pallas import tpu_sc as plsc

NUM_EXPERTS = 3
N_LAYER = 48
N_EMBD = 4096
BATCH = 16384
N_ROWS = NUM_EXPERTS * N_LAYER

NUM_CORES = 2
NUM_SUBCORES = 16
LANES = 16
NUM_WORKERS = NUM_CORES * NUM_SUBCORES
B_PER_W = BATCH // NUM_WORKERS
F_DIRECT = 448
K = 8
CHUNKS = (B_PER_W - F_DIRECT) // K
NBUF = 2
IDX_PAD = B_PER_W + LANES

_mesh = plsc.VectorSubcoreMesh(core_axis_name="c", subcore_axis_name="s")


@functools.partial(
    pl.kernel,
    mesh=_mesh,
    out_type=jax.ShapeDtypeStruct((BATCH, N_EMBD), jnp.float32),
    scratch_types=[
        pltpu.VMEM((B_PER_W,), jnp.int32),
        pltpu.VMEM((B_PER_W,), jnp.int32),
        pltpu.VMEM((IDX_PAD,), jnp.int32),
        pltpu.VMEM((NBUF, K, N_EMBD), jnp.float32),
        pltpu.VMEM_SHARED((N_ROWS * N_EMBD,), jnp.float32),
        pltpu.SemaphoreType.DMA((NBUF,)),
        pltpu.SemaphoreType.DMA((NBUF,)),
        pltpu.SemaphoreType.DMA,
    ],
)
def _lookup_kernel(table_hbm, task_hbm, layer_hbm, out_hbm,
                   task_v, layer_v, idx_v, rows_v, table_sp, gsem, wsem, dsem):
    sid = lax.axis_index("s")
    wid = sid * NUM_CORES + lax.axis_index("c")
    base = wid * B_PER_W

    n_chunks = N_ROWS // 8
    csz = 8 * N_EMBD

    def stage(j):
        off = pl.multiple_of(j * csz, 8)
        pltpu.sync_copy(table_hbm.at[pl.ds(off, csz)],
                        table_sp.at[pl.ds(off, csz)])

    stage(sid)

    @pl.when(sid < n_chunks - NUM_SUBCORES)
    def _():
        stage(NUM_SUBCORES + sid)

    pltpu.sync_copy(task_hbm.at[pl.ds(base, B_PER_W)], task_v)
    pltpu.sync_copy(layer_hbm.at[pl.ds(base, B_PER_W)], layer_v)

    def fuse(i, carry):
        sl = pl.ds(i * LANES, LANES)
        idx_v[sl] = task_v[sl] * N_LAYER + layer_v[sl]
        return carry

    lax.fori_loop(0, B_PER_W // LANES, fuse, 0)
    plsc.subcore_barrier()

    for g in range(F_DIRECT // LANES):
        dvec = idx_v[pl.ds(g * LANES, LANES)]
        for k in range(LANES):
            p = g * LANES + k
            src = table_sp.at[
                pl.ds(pl.multiple_of(dvec[k] * N_EMBD, 8), N_EMBD)]
            pltpu.make_async_copy(src, out_hbm.at[base + p], dsem).start()

    def fill(c, b):
        vec = idx_v[pl.ds(pl.multiple_of(F_DIRECT + c * K, 8), LANES)]
        for k in range(K):
            src = table_sp.at[
                pl.ds(pl.multiple_of(vec[k] * N_EMBD, 8), N_EMBD)]
            pltpu.make_async_copy(src, rows_v.at[b, k], gsem.at[b]).start()

    def fill_wait(b):
        for k in range(K):
            pltpu.make_async_copy(table_sp.at[pl.ds(0, N_EMBD)],
                                  rows_v.at[b, k], gsem.at[b]).wait()

    def write_desc(c, b):
        off = pl.multiple_of(F_DIRECT + c * K, 8)
        return pltpu.make_async_copy(rows_v.at[b],
                                     out_hbm.at[pl.ds(base + off, K)],
                                     wsem.at[b])

    for b in range(NBUF):
        fill(b, b)

    def step(c, b):
        fill_wait(b)
        wr = write_desc(c, b)
        wr.start()
        wr.wait()

        @pl.when(c + NBUF < CHUNKS)
        def _():
            fill(c + NBUF, b)

    def outer(i, carry):
        for b in range(NBUF):
            step(i * NBUF + b, b)
        return carry

    full = CHUNKS // NBUF
    lax.fori_loop(0, full, outer, 0)
    for c in range(full * NBUF, CHUNKS):
        step(c, c % NBUF)

    for g in range(F_DIRECT // LANES):
        sl = pl.ds(0, LANES * N_EMBD)
        pltpu.make_async_copy(table_hbm.at[sl], table_sp.at[sl], dsem).wait()


def kernel(experts, task_idx, layer_idx):
    table = experts.reshape(N_ROWS * N_EMBD)
    return _lookup_kernel(table,
                          task_idx.astype(jnp.int32),
                          layer_idx.astype(jnp.int32))


# speedup vs baseline: 1.6899x; 1.0010x over previous
import functools

import jax
import jax.numpy as jnp
from jax import lax
from jax.experimental import pallas as pl
from jax.experimental.pallas import tpu as pltpu
from jax.experimental.pallas import tpu_sc as plsc

NUM_EXPERTS = 3
N_LAYER = 48
N_EMBD = 4096
BATCH = 16384
N_ROWS = NUM_EXPERTS * N_LAYER

NUM_CORES = 2
NUM_SUBCORES = 16
LANES = 16
NUM_WORKERS = NUM_CORES * NUM_SUBCORES
B_PER_W = BATCH // NUM_WORKERS
F_DIRECT = 512
K = 8
CHUNKS = (B_PER_W - F_DIRECT) // K
NBUF = 2
IDX_PAD = B_PER_W + LANES

_mesh = plsc.VectorSubcoreMesh(core_axis_name="c", subcore_axis_name="s")


@functools.partial(
    pl.kernel,
    mesh=_mesh,
    out_type=jax.ShapeDtypeStruct((BATCH, N_EMBD), jnp.float32),
    scratch_types=[
        pltpu.VMEM((B_PER_W,), jnp.int32),
        pltpu.VMEM((B_PER_W,), jnp.int32),
        pltpu.VMEM((IDX_PAD,), jnp.int32),
        pltpu.VMEM((NBUF, K, N_EMBD), jnp.float32),
        pltpu.VMEM_SHARED((N_ROWS * N_EMBD,), jnp.float32),
        pltpu.SemaphoreType.DMA((NBUF,)),
        pltpu.SemaphoreType.DMA((NBUF,)),
        pltpu.SemaphoreType.DMA,
    ],
)
def _lookup_kernel(table_hbm, task_hbm, layer_hbm, out_hbm,
                   task_v, layer_v, idx_v, rows_v, table_sp, gsem, wsem, dsem):
    sid = lax.axis_index("s")
    wid = sid * NUM_CORES + lax.axis_index("c")
    base = wid * B_PER_W

    n_chunks = N_ROWS // 8
    csz = 8 * N_EMBD

    def stage(j):
        off = pl.multiple_of(j * csz, 8)
        pltpu.sync_copy(table_hbm.at[pl.ds(off, csz)],
                        table_sp.at[pl.ds(off, csz)])

    stage(sid)

    @pl.when(sid < n_chunks - NUM_SUBCORES)
    def _():
        stage(NUM_SUBCORES + sid)

    pltpu.sync_copy(task_hbm.at[pl.ds(base, B_PER_W)], task_v)
    pltpu.sync_copy(layer_hbm.at[pl.ds(base, B_PER_W)], layer_v)

    def fuse(i, carry):
        sl = pl.ds(i * LANES, LANES)
        idx_v[sl] = task_v[sl] * N_LAYER + layer_v[sl]
        return carry

    lax.fori_loop(0, B_PER_W // LANES, fuse, 0)
    plsc.subcore_barrier()

    for g in range(F_DIRECT // LANES):
        dvec = idx_v[pl.ds(g * LANES, LANES)]
        for k in range(LANES):
            p = g * LANES + k
            src = table_sp.at[
                pl.ds(pl.multiple_of(dvec[k] * N_EMBD, 8), N_EMBD)]
            pltpu.make_async_copy(src, out_hbm.at[base + p], dsem).start()

    def fill(c, b):
        vec = idx_v[pl.ds(pl.multiple_of(F_DIRECT + c * K, 8), LANES)]
        for k in range(K):
            src = table_sp.at[
                pl.ds(pl.multiple_of(vec[k] * N_EMBD, 8), N_EMBD)]
            pltpu.make_async_copy(src, rows_v.at[b, k], gsem.at[b]).start()

    def fill_wait(b):
        for k in range(K):
            pltpu.make_async_copy(table_sp.at[pl.ds(0, N_EMBD)],
                                  rows_v.at[b, k], gsem.at[b]).wait()

    def write_desc(c, b):
        off = pl.multiple_of(F_DIRECT + c * K, 8)
        return pltpu.make_async_copy(rows_v.at[b],
                                     out_hbm.at[pl.ds(base + off, K)],
                                     wsem.at[b])

    if CHUNKS > 0:
        for b in range(NBUF):
            fill(b, b)

        def step(c, b):
            fill_wait(b)
            wr = write_desc(c, b)
            wr.start()
            wr.wait()

            @pl.when(c + NBUF < CHUNKS)
            def _():
                fill(c + NBUF, b)

        def outer(i, carry):
            for b in range(NBUF):
                step(i * NBUF + b, b)
            return carry

        full = CHUNKS // NBUF
        lax.fori_loop(0, full, outer, 0)
        for c in range(full * NBUF, CHUNKS):
            step(c, c % NBUF)

    for g in range(F_DIRECT // LANES):
        sl = pl.ds(0, LANES * N_EMBD)
        pltpu.make_async_copy(table_hbm.at[sl], table_sp.at[sl], dsem).wait()


def kernel(experts, task_idx, layer_idx):
    table = experts.reshape(N_ROWS * N_EMBD)
    return _lookup_kernel(table,
                          task_idx.astype(jnp.int32),
                          layer_idx.astype(jnp.int32))
